# baseline (speedup 1.0000x reference)
import functools

import jax
import jax.numpy as jnp
from jax.experimental import pallas as pl

_J = 32
_I = 2048
_N = 16
_JN = _J * _N
_K1 = 20
_K2 = 12
_BF = jnp.bfloat16


def _squash_head(s, bias):
    ssum = jnp.sum(s, axis=1, keepdims=True)
    sb = jnp.where(ssum == 0.0, 0.0, s + bias)
    sq = jnp.sum(sb * sb, axis=1, keepdims=True)
    return (sq / (1.0 + sq)) * sb / jnp.sqrt(sq + 1e-8)


def _topk_mask(vals, k):
    jj = jax.lax.broadcasted_iota(jnp.int32, (_J, _J), 0)
    ll = jax.lax.broadcasted_iota(jnp.int32, (_J, _J), 1)
    row = jnp.sum(jnp.where(jj == ll, jnp.broadcast_to(vals, (_J, _J)), 0.0),
                  axis=0, keepdims=True)
    beats = (row > vals) | ((row == vals) & (ll < jj))
    rank = jnp.sum(beats.astype(jnp.float32), axis=1, keepdims=True)
    return rank < float(k)


def _masked_softmax(a, m):
    mx = jnp.max(jnp.where(m, a, -jnp.inf), axis=0, keepdims=True)
    e = jnp.where(m, jnp.exp(a - mx), 0.0)
    z = jnp.sum(e, axis=0, keepdims=True)
    return e / z


def _entropy_mean(c):
    lg = jnp.log(jnp.where(c > 0.0, c, 1.0))
    return -jnp.sum(c * lg) * (1.0 / _I)


def _logits(u2, v):
    vb = v.astype(_BF)
    tiled = jnp.concatenate([vb] * _J, axis=1)
    jj = jax.lax.broadcasted_iota(jnp.int32, (_J, _JN), 0)
    ll = jax.lax.broadcasted_iota(jnp.int32, (_J, _JN), 1)
    vmat = jnp.where(jj == ll // _N, tiled, jnp.array(0, _BF))
    return jax.lax.dot_general(vmat, u2, (((1,), (0,)), ((), ())),
                               preferred_element_type=jnp.float32)


def _svec(u2, c):
    d = jax.lax.dot_general(u2, c.astype(_BF), (((1,), (1,)), ((), ())),
                            preferred_element_type=jnp.float32)
    rr = jax.lax.broadcasted_iota(jnp.int32, (_JN, _J), 0)
    ll = jax.lax.broadcasted_iota(jnp.int32, (_JN, _J), 1)
    s_flat = jnp.sum(jnp.where(rr // _N == ll, d, 0.0), axis=1)
    return s_flat.reshape(_J, _N)


def _routing_kernel(ub_ref, bias_ref, v_ref, ent_ref):
    u2 = ub_ref[0].reshape(_JN, _I)
    bias = bias_ref[...]

    w0 = jnp.full((_I, 1), 1.0 / 32.0, dtype=_BF)
    s0 = jax.lax.dot_general(u2, w0, (((1,), (0,)), ((), ())),
                             preferred_element_type=jnp.float32).reshape(_J, _N)
    v0 = _squash_head(s0, bias)

    a1 = _logits(u2, v0)

    p1 = _masked_softmax(a1, jnp.full((_J, 1), True))
    m1 = _topk_mask(jnp.sum(p1, axis=1, keepdims=True) * (1.0 / _I), _K1)

    c1 = _masked_softmax(a1, m1)
    ent1 = _entropy_mean(c1)
    s1 = _svec(u2, c1)
    v1 = _squash_head(s1, bias)

    a2 = a1 + _logits(u2, v1)

    p2 = _masked_softmax(a2, m1)
    m2 = _topk_mask(jnp.sum(p2, axis=1, keepdims=True) * (1.0 / _I), _K2) & m1

    c2 = _masked_softmax(a2, m2)
    ent2 = _entropy_mean(c2)
    v_ref[0] = _squash_head(_svec(u2, c2), bias)

    lane = jax.lax.broadcasted_iota(jnp.int32, (1, 128), 1)
    ent0 = jnp.log(jnp.float32(32.0))
    ent = jnp.where(lane == 0, ent0,
                    jnp.where(lane == 1, ent1,
                              jnp.where(lane == 2, ent2, 0.0)))
    ent_ref[0] = ent


@functools.partial(jax.jit, static_argnames=())
def kernel(u_hat, iters, bias):
    del iters
    b = u_hat.shape[0]
    ub = jnp.transpose(u_hat.astype(_BF), (0, 1, 3, 2))
    v, ent = pl.pallas_call(
        _routing_kernel,
        grid=(b,),
        in_specs=[
            pl.BlockSpec((1, _J, _N, _I), lambda i: (i, 0, 0, 0)),
            pl.BlockSpec((_J, _N), lambda i: (0, 0)),
        ],
        out_specs=[
            pl.BlockSpec((1, _J, _N), lambda i: (i, 0, 0)),
            pl.BlockSpec((1, 1, 128), lambda i: (i, 0, 0)),
        ],
        out_shape=[
            jax.ShapeDtypeStruct((b, _J, _N), jnp.float32),
            jax.ShapeDtypeStruct((b, 1, 128), jnp.float32),
        ],
    )(ub, bias)
    return (v, ent.reshape(b, 128)[:, :3])


# speedup vs baseline: 1.7751x; 1.7751x over previous
import functools

import jax
import jax.numpy as jnp
from jax.experimental import pallas as pl

_J = 32
_I = 2048
_N = 16
_JN = _J * _N
_K1 = 20
_K2 = 12
_BF = jnp.bfloat16
_BB = 4


def _squash_head(s, bias):
    ssum = jnp.sum(s, axis=1, keepdims=True)
    sb = jnp.where(ssum == 0.0, 0.0, s + bias)
    sq = jnp.sum(sb * sb, axis=1, keepdims=True)
    return (sq / (1.0 + sq)) * sb / jnp.sqrt(sq + 1e-8)


def _topk_mask(vals, k):
    jj = jax.lax.broadcasted_iota(jnp.int32, (_J, _J), 0)
    ll = jax.lax.broadcasted_iota(jnp.int32, (_J, _J), 1)
    row = jnp.sum(jnp.where(jj == ll, jnp.broadcast_to(vals, (_J, _J)), 0.0),
                  axis=0, keepdims=True)
    beats = (row > vals) | ((row == vals) & (ll < jj))
    rank = jnp.sum(beats.astype(jnp.float32), axis=1, keepdims=True)
    return rank < float(k)


def _masked_softmax(a, m):
    mx = jnp.max(jnp.where(m, a, -jnp.inf), axis=0, keepdims=True)
    e = jnp.where(m, jnp.exp(a - mx), 0.0)
    z = jnp.sum(e, axis=0, keepdims=True)
    return e / z


def _entropy_mean(c):
    lg = jnp.log(jnp.where(c > 0.0, c, 1.0))
    return -jnp.sum(c * lg) * (1.0 / _I)


def _logits(u2, v):
    vb = v.astype(_BF)
    tiled = jnp.concatenate([vb] * _J, axis=1)
    jj = jax.lax.broadcasted_iota(jnp.int32, (_J, _JN), 0)
    ll = jax.lax.broadcasted_iota(jnp.int32, (_J, _JN), 1)
    vmat = jnp.where(jj == ll // _N, tiled, jnp.array(0, _BF))
    return jax.lax.dot_general(vmat, u2, (((1,), (0,)), ((), ())),
                               preferred_element_type=jnp.float32)


def _svec(u2, c):
    d = jax.lax.dot_general(u2, c.astype(_BF), (((1,), (1,)), ((), ())),
                            preferred_element_type=jnp.float32)
    rr = jax.lax.broadcasted_iota(jnp.int32, (_JN, _J), 0)
    ll = jax.lax.broadcasted_iota(jnp.int32, (_JN, _J), 1)
    s_flat = jnp.sum(jnp.where(rr // _N == ll, d, 0.0), axis=1)
    return s_flat.reshape(_J, _N)


def _routing_kernel(ub_ref, bias_ref, v_ref, ent_ref):
    bias = bias_ref[...]
    lane = jax.lax.broadcasted_iota(jnp.int32, (1, 128), 1)
    ent0 = jnp.log(jnp.float32(32.0))
    w0 = jnp.full((_I, 1), 1.0 / 32.0, dtype=_BF)
    R = range(_BB)

    u2 = [ub_ref[k].reshape(_JN, _I) for k in R]

    s0 = [jax.lax.dot_general(u2[k], w0, (((1,), (0,)), ((), ())),
                              preferred_element_type=jnp.float32
                              ).reshape(_J, _N) for k in R]
    v0 = [_squash_head(s0[k], bias) for k in R]

    a1 = [_logits(u2[k], v0[k]) for k in R]

    ones = jnp.full((_J, 1), True)
    p1 = [_masked_softmax(a1[k], ones) for k in R]
    m1 = [_topk_mask(jnp.sum(p1[k], axis=1, keepdims=True) * (1.0 / _I), _K1)
          for k in R]

    c1 = [_masked_softmax(a1[k], m1[k]) for k in R]
    ent1 = [_entropy_mean(c1[k]) for k in R]
    s1 = [_svec(u2[k], c1[k]) for k in R]
    v1 = [_squash_head(s1[k], bias) for k in R]

    a2 = [a1[k] + _logits(u2[k], v1[k]) for k in R]

    p2 = [_masked_softmax(a2[k], m1[k]) for k in R]
    m2 = [_topk_mask(jnp.sum(p2[k], axis=1, keepdims=True) * (1.0 / _I), _K2)
          & m1[k] for k in R]

    c2 = [_masked_softmax(a2[k], m2[k]) for k in R]
    ent2 = [_entropy_mean(c2[k]) for k in R]
    s2 = [_svec(u2[k], c2[k]) for k in R]
    for k in R:
        v_ref[k] = _squash_head(s2[k], bias)
        ent = jnp.where(lane == 0, ent0,
                        jnp.where(lane == 1, ent1[k],
                                  jnp.where(lane == 2, ent2[k], 0.0)))
        ent_ref[k] = ent.reshape(1, 1, 128)[0]


@functools.partial(jax.jit, static_argnames=())
def kernel(u_hat, iters, bias):
    del iters
    b = u_hat.shape[0]
    ub = jnp.transpose(u_hat.astype(_BF), (0, 1, 3, 2))
    v, ent = pl.pallas_call(
        _routing_kernel,
        grid=(b // _BB,),
        in_specs=[
            pl.BlockSpec((_BB, _J, _N, _I), lambda i: (i, 0, 0, 0)),
            pl.BlockSpec((_J, _N), lambda i: (0, 0)),
        ],
        out_specs=[
            pl.BlockSpec((_BB, _J, _N), lambda i: (i, 0, 0)),
            pl.BlockSpec((_BB, 1, 128), lambda i: (i, 0, 0)),
        ],
        out_shape=[
            jax.ShapeDtypeStruct((b, _J, _N), jnp.float32),
            jax.ShapeDtypeStruct((b, 1, 128), jnp.float32),
        ],
    )(ub, bias)
    return (v, ent.reshape(b, 128)[:, :3])


# speedup vs baseline: 1.8219x; 1.0263x over previous
import functools

import jax
import jax.numpy as jnp
from jax.experimental import pallas as pl

_J = 32
_I = 2048
_N = 16
_JN = _J * _N
_K1 = 20
_K2 = 12
_BF = jnp.bfloat16
_BB = 8


def _squash_head(s, bias):
    ssum = jnp.sum(s, axis=1, keepdims=True)
    sb = jnp.where(ssum == 0.0, 0.0, s + bias)
    sq = jnp.sum(sb * sb, axis=1, keepdims=True)
    return (sq / (1.0 + sq)) * sb / jnp.sqrt(sq + 1e-8)


def _topk_mask(vals, k):
    jj = jax.lax.broadcasted_iota(jnp.int32, (_J, _J), 0)
    ll = jax.lax.broadcasted_iota(jnp.int32, (_J, _J), 1)
    row = jnp.sum(jnp.where(jj == ll, jnp.broadcast_to(vals, (_J, _J)), 0.0),
                  axis=0, keepdims=True)
    beats = (row > vals) | ((row == vals) & (ll < jj))
    rank = jnp.sum(beats.astype(jnp.float32), axis=1, keepdims=True)
    return rank < float(k)


def _masked_softmax(a, m):
    mx = jnp.max(jnp.where(m, a, -jnp.inf), axis=0, keepdims=True)
    e = jnp.where(m, jnp.exp(a - mx), 0.0)
    z = jnp.sum(e, axis=0, keepdims=True)
    return e / z


def _entropy_mean(c):
    lg = jnp.log(jnp.where(c > 0.0, c, 1.0))
    return -jnp.sum(c * lg) * (1.0 / _I)


def _logits(u2, v):
    vb = v.astype(_BF)
    tiled = jnp.concatenate([vb] * _J, axis=1)
    jj = jax.lax.broadcasted_iota(jnp.int32, (_J, _JN), 0)
    ll = jax.lax.broadcasted_iota(jnp.int32, (_J, _JN), 1)
    vmat = jnp.where(jj == ll // _N, tiled, jnp.array(0, _BF))
    return jax.lax.dot_general(vmat, u2, (((1,), (0,)), ((), ())),
                               preferred_element_type=jnp.float32)


def _svec(u2, c):
    d = jax.lax.dot_general(u2, c.astype(_BF), (((1,), (1,)), ((), ())),
                            preferred_element_type=jnp.float32)
    rr = jax.lax.broadcasted_iota(jnp.int32, (_JN, _J), 0)
    ll = jax.lax.broadcasted_iota(jnp.int32, (_JN, _J), 1)
    s_flat = jnp.sum(jnp.where(rr // _N == ll, d, 0.0), axis=1)
    return s_flat.reshape(_J, _N)


def _routing_kernel(ub_ref, bias_ref, v_ref, ent_ref):
    bias = bias_ref[...]
    lane = jax.lax.broadcasted_iota(jnp.int32, (1, 128), 1)
    ent0 = jnp.log(jnp.float32(32.0))
    w0 = jnp.full((_I, 1), 1.0 / 32.0, dtype=_BF)
    R = range(_BB)

    u2 = [ub_ref[k].reshape(_JN, _I) for k in R]

    s0 = [jax.lax.dot_general(u2[k], w0, (((1,), (0,)), ((), ())),
                              preferred_element_type=jnp.float32
                              ).reshape(_J, _N) for k in R]
    v0 = [_squash_head(s0[k], bias) for k in R]

    a1 = [_logits(u2[k], v0[k]) for k in R]

    ones = jnp.full((_J, 1), True)
    p1 = [_masked_softmax(a1[k], ones) for k in R]
    m1 = [_topk_mask(jnp.sum(p1[k], axis=1, keepdims=True) * (1.0 / _I), _K1)
          for k in R]

    c1 = [_masked_softmax(a1[k], m1[k]) for k in R]
    ent1 = [_entropy_mean(c1[k]) for k in R]
    s1 = [_svec(u2[k], c1[k]) for k in R]
    v1 = [_squash_head(s1[k], bias) for k in R]

    a2 = [a1[k] + _logits(u2[k], v1[k]) for k in R]

    p2 = [_masked_softmax(a2[k], m1[k]) for k in R]
    m2 = [_topk_mask(jnp.sum(p2[k], axis=1, keepdims=True) * (1.0 / _I), _K2)
          & m1[k] for k in R]

    c2 = [_masked_softmax(a2[k], m2[k]) for k in R]
    ent2 = [_entropy_mean(c2[k]) for k in R]
    s2 = [_svec(u2[k], c2[k]) for k in R]
    for k in R:
        v_ref[k] = _squash_head(s2[k], bias)
        ent = jnp.where(lane == 0, ent0,
                        jnp.where(lane == 1, ent1[k],
                                  jnp.where(lane == 2, ent2[k], 0.0)))
        ent_ref[k] = ent.reshape(1, 1, 128)[0]


@functools.partial(jax.jit, static_argnames=())
def kernel(u_hat, iters, bias):
    del iters
    b = u_hat.shape[0]
    ub = jnp.transpose(u_hat.astype(_BF), (0, 1, 3, 2))
    v, ent = pl.pallas_call(
        _routing_kernel,
        grid=(b // _BB,),
        in_specs=[
            pl.BlockSpec((_BB, _J, _N, _I), lambda i: (i, 0, 0, 0)),
            pl.BlockSpec((_J, _N), lambda i: (0, 0)),
        ],
        out_specs=[
            pl.BlockSpec((_BB, _J, _N), lambda i: (i, 0, 0)),
            pl.BlockSpec((_BB, 1, 128), lambda i: (i, 0, 0)),
        ],
        out_shape=[
            jax.ShapeDtypeStruct((b, _J, _N), jnp.float32),
            jax.ShapeDtypeStruct((b, 1, 128), jnp.float32),
        ],
    )(ub, bias)
    return (v, ent.reshape(b, 128)[:, :3])


# speedup vs baseline: 1.8244x; 1.0014x over previous
import functools

import jax
import jax.numpy as jnp
from jax.experimental import pallas as pl

_J = 32
_I = 2048
_N = 16
_JN = _J * _N
_K1 = 20
_K2 = 12
_BF = jnp.bfloat16
_BB = 8


def _squash_head(s, bias):
    ssum = jnp.sum(s, axis=1, keepdims=True)
    sb = jnp.where(ssum == 0.0, 0.0, s + bias)
    sq = jnp.sum(sb * sb, axis=1, keepdims=True)
    return (sq / (1.0 + sq)) * sb / jnp.sqrt(sq + 1e-8)


def _topk_mask(vals, k):
    jj = jax.lax.broadcasted_iota(jnp.int32, (_J, _J), 0)
    ll = jax.lax.broadcasted_iota(jnp.int32, (_J, _J), 1)
    row = jnp.sum(jnp.where(jj == ll, jnp.broadcast_to(vals, (_J, _J)), 0.0),
                  axis=0, keepdims=True)
    beats = (row > vals) | ((row == vals) & (ll < jj))
    rank = jnp.sum(beats.astype(jnp.float32), axis=1, keepdims=True)
    return rank < float(k)


def _masked_softmax(a, m):
    mx = jnp.max(jnp.where(m, a, -jnp.inf), axis=0, keepdims=True)
    e = jnp.where(m, jnp.exp(a - mx), 0.0)
    z = jnp.sum(e, axis=0, keepdims=True)
    return e / z


def _entropy_mean(c):
    lg = jnp.log(jnp.where(c > 0.0, c, 1.0))
    return -jnp.sum(c * lg) * (1.0 / _I)


def _logits(u2, v):
    vb = v.astype(_BF)
    tiled = jnp.concatenate([vb] * _J, axis=1)
    jj = jax.lax.broadcasted_iota(jnp.int32, (_J, _JN), 0)
    ll = jax.lax.broadcasted_iota(jnp.int32, (_J, _JN), 1)
    vmat = jnp.where(jj == ll // _N, tiled, jnp.array(0, _BF))
    return jax.lax.dot_general(vmat, u2, (((1,), (0,)), ((), ())),
                               preferred_element_type=jnp.float32)


def _svec(u2, c):
    d = jax.lax.dot_general(u2, c.astype(_BF), (((1,), (1,)), ((), ())),
                            preferred_element_type=jnp.float32)
    rr = jax.lax.broadcasted_iota(jnp.int32, (_JN, _J), 0)
    ll = jax.lax.broadcasted_iota(jnp.int32, (_JN, _J), 1)
    s_flat = jnp.sum(jnp.where(rr // _N == ll, d, 0.0), axis=1)
    return s_flat.reshape(_J, _N)


def _routing_kernel(ub_ref, bias_ref, v_ref, ent_ref):
    bias = bias_ref[...]
    lane = jax.lax.broadcasted_iota(jnp.int32, (1, 128), 1)
    ent0 = jnp.log(jnp.float32(32.0))
    w0 = jnp.full((_I, 1), 1.0 / 32.0, dtype=_BF)
    R = range(_BB)

    u2 = [ub_ref[k].reshape(_JN, _I) for k in R]

    s0 = [jax.lax.dot_general(u2[k], w0, (((1,), (0,)), ((), ())),
                              preferred_element_type=jnp.float32
                              ).reshape(_J, _N) for k in R]
    v0 = [_squash_head(s0[k], bias) for k in R]

    a1 = [_logits(u2[k], v0[k]) for k in R]

    ones = jnp.full((_J, 1), True)
    p1 = [_masked_softmax(a1[k], ones) for k in R]
    m1 = [_topk_mask(jnp.sum(p1[k], axis=1, keepdims=True) * (1.0 / _I), _K1)
          for k in R]

    c1 = [_masked_softmax(a1[k], m1[k]) for k in R]
    ent1 = [_entropy_mean(c1[k]) for k in R]
    s1 = [_svec(u2[k], c1[k]) for k in R]
    v1 = [_squash_head(s1[k], bias) for k in R]

    a2 = [a1[k] + _logits(u2[k], v1[k]) for k in R]

    mx2 = [jnp.max(jnp.where(m1[k], a2[k], -jnp.inf), axis=0, keepdims=True)
           for k in R]
    e2 = [jnp.where(m1[k], jnp.exp(a2[k] - mx2[k]), 0.0) for k in R]
    z2 = [jnp.sum(e2[k], axis=0, keepdims=True) for k in R]
    p2 = [e2[k] / z2[k] for k in R]
    m2 = [_topk_mask(jnp.sum(p2[k], axis=1, keepdims=True) * (1.0 / _I), _K2)
          & m1[k] for k in R]

    e2m = [jnp.where(m2[k], e2[k], 0.0) for k in R]
    c2 = [e2m[k] / jnp.sum(e2m[k], axis=0, keepdims=True) for k in R]
    ent2 = [_entropy_mean(c2[k]) for k in R]
    s2 = [_svec(u2[k], c2[k]) for k in R]
    for k in R:
        v_ref[k] = _squash_head(s2[k], bias)
        ent = jnp.where(lane == 0, ent0,
                        jnp.where(lane == 1, ent1[k],
                                  jnp.where(lane == 2, ent2[k], 0.0)))
        ent_ref[k] = ent.reshape(1, 1, 128)[0]


@functools.partial(jax.jit, static_argnames=())
def kernel(u_hat, iters, bias):
    del iters
    b = u_hat.shape[0]
    ub = jnp.transpose(u_hat.astype(_BF), (0, 1, 3, 2))
    v, ent = pl.pallas_call(
        _routing_kernel,
        grid=(b // _BB,),
        in_specs=[
            pl.BlockSpec((_BB, _J, _N, _I), lambda i: (i, 0, 0, 0)),
            pl.BlockSpec((_J, _N), lambda i: (0, 0)),
        ],
        out_specs=[
            pl.BlockSpec((_BB, _J, _N), lambda i: (i, 0, 0)),
            pl.BlockSpec((_BB, 1, 128), lambda i: (i, 0, 0)),
        ],
        out_shape=[
            jax.ShapeDtypeStruct((b, _J, _N), jnp.float32),
            jax.ShapeDtypeStruct((b, 1, 128), jnp.float32),
        ],
    )(ub, bias)
    return (v, ent.reshape(b, 128)[:, :3])


# speedup vs baseline: 1.9009x; 1.0419x over previous
import functools

import jax
import jax.numpy as jnp
from jax.experimental import pallas as pl

_J = 32
_I = 2048
_N = 16
_JN = _J * _N
_K1 = 20
_K2 = 12
_BF = jnp.bfloat16
_BB = 8


def _squash_head(s, bias):
    ssum = jnp.sum(s, axis=1, keepdims=True)
    sb = jnp.where(ssum == 0.0, 0.0, s + bias)
    sq = jnp.sum(sb * sb, axis=1, keepdims=True)
    return (sq / (1.0 + sq)) * sb / jnp.sqrt(sq + 1e-8)


def _topk_mask(vals, k):
    jj = jax.lax.broadcasted_iota(jnp.int32, (_J, _J), 0)
    ll = jax.lax.broadcasted_iota(jnp.int32, (_J, _J), 1)
    row = jnp.sum(jnp.where(jj == ll, jnp.broadcast_to(vals, (_J, _J)), 0.0),
                  axis=0, keepdims=True)
    beats = (row > vals) | ((row == vals) & (ll < jj))
    rank = jnp.sum(beats.astype(jnp.float32), axis=1, keepdims=True)
    return rank < float(k)


def _masked_softmax(a, m):
    mx = jnp.max(jnp.where(m, a, -jnp.inf), axis=0, keepdims=True)
    e = jnp.where(m, jnp.exp(a - mx), 0.0)
    z = jnp.sum(e, axis=0, keepdims=True)
    return e / z


def _entropy_mean(c):
    lg = jnp.log(jnp.where(c > 0.0, c, 1.0))
    return -jnp.sum(c * lg) * (1.0 / _I)


def _logits(u2, v):
    vb = v.astype(_BF)
    tiled = jnp.concatenate([vb] * _J, axis=1)
    jj = jax.lax.broadcasted_iota(jnp.int32, (_J, _JN), 0)
    ll = jax.lax.broadcasted_iota(jnp.int32, (_J, _JN), 1)
    vmat = jnp.where(jj == ll // _N, tiled, jnp.array(0, _BF))
    return jax.lax.dot_general(vmat, u2, (((1,), (0,)), ((), ())),
                               preferred_element_type=jnp.float32)


def _svec(u2, c):
    d = jax.lax.dot_general(u2, c.astype(_BF), (((1,), (1,)), ((), ())),
                            preferred_element_type=jnp.float32)
    rr = jax.lax.broadcasted_iota(jnp.int32, (_JN, _J), 0)
    ll = jax.lax.broadcasted_iota(jnp.int32, (_JN, _J), 1)
    s_flat = jnp.sum(jnp.where(rr // _N == ll, d, 0.0), axis=1)
    return s_flat.reshape(_J, _N)


def _routing_kernel(ub_ref, bias_ref, v_ref, ent_ref):
    bias = bias_ref[...]
    lane = jax.lax.broadcasted_iota(jnp.int32, (1, 128), 1)
    ent0 = jnp.log(jnp.float32(32.0))
    R = range(_BB)

    u2 = [ub_ref[k].reshape(_JN, _I) for k in R]

    s0 = [jnp.sum(u2[k].astype(jnp.float32), axis=1, keepdims=True
                  ).reshape(_J, _N) * (1.0 / 32.0) for k in R]
    v0 = [_squash_head(s0[k], bias) for k in R]

    a1 = [_logits(u2[k], v0[k]) for k in R]

    ones = jnp.full((_J, 1), True)
    p1 = [_masked_softmax(a1[k], ones) for k in R]
    m1 = [_topk_mask(jnp.sum(p1[k], axis=1, keepdims=True) * (1.0 / _I), _K1)
          for k in R]

    c1 = [_masked_softmax(a1[k], m1[k]) for k in R]
    ent1 = [_entropy_mean(c1[k]) for k in R]
    s1 = [_svec(u2[k], c1[k]) for k in R]
    v1 = [_squash_head(s1[k], bias) for k in R]

    a2 = [a1[k] + _logits(u2[k], v1[k]) for k in R]

    mx2 = [jnp.max(jnp.where(m1[k], a2[k], -jnp.inf), axis=0, keepdims=True)
           for k in R]
    e2 = [jnp.where(m1[k], jnp.exp(a2[k] - mx2[k]), 0.0) for k in R]
    z2 = [jnp.sum(e2[k], axis=0, keepdims=True) for k in R]
    p2 = [e2[k] / z2[k] for k in R]
    m2 = [_topk_mask(jnp.sum(p2[k], axis=1, keepdims=True) * (1.0 / _I), _K2)
          & m1[k] for k in R]

    e2m = [jnp.where(m2[k], e2[k], 0.0) for k in R]
    c2 = [e2m[k] / jnp.sum(e2m[k], axis=0, keepdims=True) for k in R]
    ent2 = [_entropy_mean(c2[k]) for k in R]
    s2 = [_svec(u2[k], c2[k]) for k in R]
    for k in R:
        v_ref[k] = _squash_head(s2[k], bias)
        ent = jnp.where(lane == 0, ent0,
                        jnp.where(lane == 1, ent1[k],
                                  jnp.where(lane == 2, ent2[k], 0.0)))
        ent_ref[k] = ent.reshape(1, 1, 128)[0]


@functools.partial(jax.jit, static_argnames=())
def kernel(u_hat, iters, bias):
    del iters
    b = u_hat.shape[0]
    ub = jnp.transpose(u_hat.astype(_BF), (0, 1, 3, 2))
    v, ent = pl.pallas_call(
        _routing_kernel,
        grid=(b // _BB,),
        in_specs=[
            pl.BlockSpec((_BB, _J, _N, _I), lambda i: (i, 0, 0, 0)),
            pl.BlockSpec((_J, _N), lambda i: (0, 0)),
        ],
        out_specs=[
            pl.BlockSpec((_BB, _J, _N), lambda i: (i, 0, 0)),
            pl.BlockSpec((_BB, 1, 128), lambda i: (i, 0, 0)),
        ],
        out_shape=[
            jax.ShapeDtypeStruct((b, _J, _N), jnp.float32),
            jax.ShapeDtypeStruct((b, 1, 128), jnp.float32),
        ],
    )(ub, bias)
    return (v, ent.reshape(b, 128)[:, :3])
